# Initial kernel scaffold; baseline (speedup 1.0000x reference)
import jax
import jax.numpy as jnp
from jax.experimental import pallas as pl


def kernel(tokens, W):
    raise NotImplementedError("write your pallas kernel here")


# speedup vs baseline: 1.4765x; 1.4765x over previous
import functools

import jax
import jax.numpy as jnp
from jax import lax
from jax.experimental import pallas as pl
from jax.experimental.pallas import tpu as pltpu
from jax.experimental.pallas import tpu_sc as plsc

EMB_DIM = 32
NUM_WORKERS = 32
CHUNK = 1600


@functools.lru_cache(maxsize=None)
def _build_gather(B: int):
    b_per_w = B // NUM_WORKERS
    n_chunks = b_per_w // CHUNK
    mesh = plsc.VectorSubcoreMesh(core_axis_name="c", subcore_axis_name="s")

    @functools.partial(
        pl.kernel,
        mesh=mesh,
        out_type=jax.ShapeDtypeStruct((B, EMB_DIM), jnp.float32),
        scratch_types=[
            pltpu.VMEM((CHUNK,), jnp.int32),
            pltpu.VMEM((CHUNK, EMB_DIM), jnp.float32),
            pltpu.SemaphoreType.DMA,
        ],
        compiler_params=pltpu.CompilerParams(use_tc_tiling_on_sc=False),
    )
    def gather_kernel(table_hbm, idx_hbm, out_hbm, idx_v, rows_v, sem):
        wid = lax.axis_index("s") * 2 + lax.axis_index("c")
        base = wid * b_per_w

        def body(i, carry):
            off = base + i * CHUNK
            pltpu.sync_copy(idx_hbm.at[pl.ds(off, CHUNK)], idx_v)
            pltpu.async_copy(table_hbm.at[idx_v], rows_v, sem).wait()
            pltpu.sync_copy(rows_v, out_hbm.at[pl.ds(off, CHUNK)])
            return carry

        lax.fori_loop(0, n_chunks, body, 0)

    return gather_kernel


def kernel(tokens, W):
    S, T = tokens.shape
    flat = tokens.reshape(S * T).astype(jnp.int32)
    out = _build_gather(S * T)(W, flat)
    return out.reshape(S, T, EMB_DIM)


# speedup vs baseline: 1.4911x; 1.0099x over previous
import functools

import jax
import jax.numpy as jnp
from jax import lax
from jax.experimental import pallas as pl
from jax.experimental.pallas import tpu as pltpu
from jax.experimental.pallas import tpu_sc as plsc

EMB_DIM = 32
NUM_WORKERS = 32
CHUNK = 1600


@functools.lru_cache(maxsize=None)
def _build_gather(B: int):
    b_per_w = B // NUM_WORKERS
    n_chunks = b_per_w // CHUNK
    mesh = plsc.VectorSubcoreMesh(core_axis_name="c", subcore_axis_name="s")

    @functools.partial(
        pl.kernel,
        mesh=mesh,
        out_type=jax.ShapeDtypeStruct((B, EMB_DIM), jnp.float32),
        scratch_types=[
            pltpu.VMEM((b_per_w,), jnp.int32),
            pltpu.VMEM((CHUNK, EMB_DIM), jnp.float32),
            pltpu.VMEM((CHUNK, EMB_DIM), jnp.float32),
            pltpu.SemaphoreType.DMA,
            pltpu.SemaphoreType.DMA,
            pltpu.SemaphoreType.DMA,
            pltpu.SemaphoreType.DMA,
        ],
        compiler_params=pltpu.CompilerParams(use_tc_tiling_on_sc=False),
    )
    def gather_kernel(table_hbm, idx_hbm, out_hbm, idx_all, rows0, rows1,
                      sg0, sg1, so0, so1):
        wid = lax.axis_index("s") * 2 + lax.axis_index("c")
        base = wid * b_per_w
        pltpu.sync_copy(idx_hbm.at[pl.ds(base, b_per_w)], idx_all)

        rows = (rows0, rows1)
        sg = (sg0, sg1)
        so = (so0, so1)

        def gather_start(i):
            b = i % 2
            return pltpu.async_copy(
                table_hbm.at[idx_all.at[pl.ds(i * CHUNK, CHUNK)]],
                rows[b], sg[b])

        def store_start(i):
            b = i % 2
            return pltpu.async_copy(
                rows[b], out_hbm.at[pl.ds(base + i * CHUNK, CHUNK)], so[b])

        g_descs = [None] * n_chunks
        o_descs = [None] * n_chunks
        g_descs[0] = gather_start(0)
        for i in range(n_chunks):
            g_descs[i].wait()
            o_descs[i] = store_start(i)
            if i + 1 < n_chunks:
                if i >= 1:
                    o_descs[i - 1].wait()
                g_descs[i + 1] = gather_start(i + 1)
        if n_chunks >= 2:
            o_descs[n_chunks - 2].wait()
        o_descs[n_chunks - 1].wait()

    return gather_kernel


def kernel(tokens, W):
    S, T = tokens.shape
    flat = tokens.reshape(S * T).astype(jnp.int32)
    out = _build_gather(S * T)(W, flat)
    return out.reshape(S, T, EMB_DIM)
